# baseline (speedup 1.0000x reference)
import functools

import jax
import jax.numpy as jnp
from jax import lax
from jax.experimental import pallas as pl
from jax.experimental.pallas import tpu as pltpu
from jax.experimental.pallas import tpu_sc as plsc

BATCH = 16384
DIM = 16
ROWS_PER_BLOCK = 8

_info = plsc.get_sparse_core_info()
_NC, _NS, _L = _info.num_cores, _info.num_subcores, _info.num_lanes
_NW = _NC * _NS
_B_PER_W = BATCH // _NW
_GROUPS = _B_PER_W // _L


def _make_gather():
    mesh = plsc.VectorSubcoreMesh(core_axis_name="c", subcore_axis_name="s")

    @functools.partial(
        pl.kernel,
        mesh=mesh,
        out_type=jax.ShapeDtypeStruct((BATCH * DIM,), jnp.float32),
        scratch_types=[
            pltpu.VMEM((_B_PER_W,), jnp.int32),
            pltpu.VMEM((_B_PER_W,), jnp.int32),
            pltpu.VMEM((_B_PER_W, 128), jnp.float32),
            pltpu.VMEM((_B_PER_W * DIM,), jnp.float32),
            pltpu.SemaphoreType.DMA,
        ],
        compiler_params=pltpu.CompilerParams(needs_layout_passes=False),
    )
    def gather_kernel(idx_hbm, table_hbm, out_hbm, idx_v, blk_v, rows_v,
                      out_v, sem):
        wid = lax.axis_index("s") * _NC + lax.axis_index("c")
        base = wid * _B_PER_W
        pltpu.sync_copy(idx_hbm.at[pl.ds(base, _B_PER_W)], idx_v)

        def compute_blocks(g):
            v = idx_v[pl.ds(g * _L, _L)]
            blk_v[pl.ds(g * _L, _L)] = v >> 3
        pl.loop(0, _GROUPS)(compute_blocks)

        pltpu.async_copy(table_hbm.at[blk_v], rows_v, sem).wait()

        lanes = lax.iota(jnp.int32, _L)

        def extract(g):
            v = idx_v[pl.ds(g * _L, _L)]
            sub = (v & 7) * DIM
            rowsel = g * _L + lanes
            outbase = (g * _L + lanes) * DIM
            for d in range(DIM):
                vals = plsc.load_gather(rows_v, [rowsel, sub + d])
                plsc.store_scatter(out_v, [outbase + d], vals)
        pl.loop(0, _GROUPS)(extract)

        pltpu.sync_copy(out_v, out_hbm.at[pl.ds(base * DIM, _B_PER_W * DIM)])

    return gather_kernel


_gather = _make_gather()


def kernel(idx, context_hat):
    idx_flat = idx.reshape(BATCH).astype(jnp.int32)
    table_blocks = context_hat.reshape(-1, ROWS_PER_BLOCK * DIM)
    out_flat = _gather(idx_flat, table_blocks)
    return out_flat.reshape(BATCH, DIM)


# speedup vs baseline: 6.0621x; 6.0621x over previous
import functools

import jax
import jax.numpy as jnp
from jax import lax
from jax.experimental import pallas as pl
from jax.experimental.pallas import tpu as pltpu
from jax.experimental.pallas import tpu_sc as plsc

BATCH = 16384
DIM = 16
TASKS = 1000000

_info = plsc.get_sparse_core_info()
_NC, _NS, _L = _info.num_cores, _info.num_subcores, _info.num_lanes
_NW = _NC * _NS
_B_PER_W = BATCH // _NW
_WAVE = 32
_N_WAVES = _B_PER_W // _WAVE


def _make_gather():
    mesh = plsc.VectorSubcoreMesh(core_axis_name="c", subcore_axis_name="s")

    @functools.partial(
        pl.kernel,
        mesh=mesh,
        out_type=jax.ShapeDtypeStruct((DIM, BATCH), jnp.float32),
        scratch_types=[
            pltpu.VMEM((_B_PER_W,), jnp.int32),
            pltpu.VMEM((_WAVE, DIM, 128), jnp.float32),
            pltpu.VMEM((DIM, _B_PER_W), jnp.float32),
            pltpu.SemaphoreType.DMA,
        ],
        compiler_params=pltpu.CompilerParams(needs_layout_passes=False),
    )
    def gather_kernel(idx_hbm, tableT_hbm, outT_hbm, idx_v, blocks_v, rows_v,
                      sem):
        wid = lax.axis_index("s") * _NC + lax.axis_index("c")
        base = wid * _B_PER_W
        pltpu.sync_copy(idx_hbm.at[pl.ds(base, _B_PER_W)], idx_v)

        lanes = lax.iota(jnp.int32, _L)

        def wave(w):
            vecs = [idx_v[pl.ds(w * _WAVE + g * _L, _L)]
                    for g in range(_WAVE // _L)]
            for t in range(_WAVE):
                v = vecs[t // _L]
                col = pl.multiple_of(v[t % _L] & -128, 128)
                pltpu.async_copy(
                    tableT_hbm.at[:, pl.ds(col, 128)],
                    blocks_v.at[t],
                    sem,
                )
            for t in range(_WAVE):
                pltpu.make_async_copy(
                    tableT_hbm.at[:, pl.ds(0, 128)],
                    blocks_v.at[t],
                    sem,
                ).wait()
            for g in range(_WAVE // _L):
                lvec = vecs[g] & 127
                tvec = jnp.full((_L,), g * _L, jnp.int32) + lanes
                for j in range(DIM):
                    vals = plsc.load_gather(
                        blocks_v,
                        [tvec, jnp.full((_L,), j, jnp.int32), lvec],
                    )
                    plsc.store_scatter(
                        rows_v,
                        [jnp.full((_L,), j, jnp.int32),
                         jnp.full((_L,), w * _WAVE + g * _L, jnp.int32) + lanes],
                        vals,
                    )
        pl.loop(0, _N_WAVES)(wave)

        pltpu.sync_copy(rows_v, outT_hbm.at[:, pl.ds(base, _B_PER_W)])

    return gather_kernel


_gather = _make_gather()


def kernel(idx, context_hat):
    idx_flat = idx.reshape(BATCH).astype(jnp.int32)
    out_t = _gather(idx_flat, context_hat.T)
    return out_t.T


# speedup vs baseline: 7.3688x; 1.2156x over previous
import functools

import jax
import jax.numpy as jnp
from jax import lax
from jax.experimental import pallas as pl
from jax.experimental.pallas import tpu as pltpu
from jax.experimental.pallas import tpu_sc as plsc

BATCH = 16384
DIM = 16
TASKS = 1000000

_info = plsc.get_sparse_core_info()
_NC, _NS, _L = _info.num_cores, _info.num_subcores, _info.num_lanes
_NW = _NC * _NS
_B_PER_W = BATCH // _NW
_WAVE = 16
_N_WAVES = _B_PER_W // _WAVE


def _make_gather():
    mesh = plsc.VectorSubcoreMesh(core_axis_name="c", subcore_axis_name="s")

    @functools.partial(
        pl.kernel,
        mesh=mesh,
        out_type=jax.ShapeDtypeStruct((DIM, BATCH), jnp.float32),
        scratch_types=[
            pltpu.VMEM((_B_PER_W,), jnp.int32),
            pltpu.VMEM((2, _WAVE, DIM, 128), jnp.float32),
            pltpu.VMEM((DIM, _B_PER_W), jnp.float32),
            pltpu.SemaphoreType.DMA,
        ],
        compiler_params=pltpu.CompilerParams(needs_layout_passes=False),
    )
    def gather_kernel(idx_hbm, tableT_hbm, outT_hbm, idx_v, blocks_v, rows_v,
                      sem):
        wid = lax.axis_index("s") * _NC + lax.axis_index("c")
        base = wid * _B_PER_W
        pltpu.sync_copy(idx_hbm.at[pl.ds(base, _B_PER_W)], idx_v)

        lanes = lax.iota(jnp.int32, _L)

        def fire(w, buf):
            for g in range(_WAVE // _L):
                v = idx_v[pl.ds(w * _WAVE + g * _L, _L)]
                for k in range(_L):
                    col = pl.multiple_of(v[k] & -128, 128)
                    pltpu.async_copy(
                        tableT_hbm.at[:, pl.ds(col, 128)],
                        blocks_v.at[buf, g * _L + k],
                        sem,
                    )

        def drain(buf):
            for t in range(_WAVE):
                pltpu.make_async_copy(
                    tableT_hbm.at[:, pl.ds(0, 128)],
                    blocks_v.at[buf, t],
                    sem,
                ).wait()

        def extract(w, buf):
            for g in range(_WAVE // _L):
                vec = idx_v[pl.ds(w * _WAVE + g * _L, _L)]
                lvec = vec & 127
                tvec = jnp.full((_L,), g * _L, jnp.int32) + lanes
                for j in range(DIM):
                    vals = plsc.load_gather(
                        blocks_v,
                        [jnp.full((_L,), buf, jnp.int32), tvec,
                         jnp.full((_L,), j, jnp.int32), lvec],
                    )
                    plsc.store_scatter(
                        rows_v,
                        [jnp.full((_L,), j, jnp.int32),
                         (w * _WAVE + g * _L) + lanes],
                        vals,
                    )

        fire(0, 0)
        def step(i):
            @pl.when(i < _N_WAVES - 1)
            def _():
                fire(i + 1, (i + 1) % 2)
            drain(i % 2)
            extract(i, i % 2)
        pl.loop(0, _N_WAVES)(step)

        pltpu.sync_copy(rows_v, outT_hbm.at[:, pl.ds(base, _B_PER_W)])

    return gather_kernel


_gather = _make_gather()


def kernel(idx, context_hat):
    idx_flat = idx.reshape(BATCH).astype(jnp.int32)
    out_t = _gather(idx_flat, context_hat.T)
    return out_t.T
